# baseline (speedup 1.0000x reference)
import functools

import jax
import jax.numpy as jnp
from jax import lax
from jax.experimental import pallas as pl
from jax.experimental.pallas import tpu as pltpu
from jax.experimental.pallas import tpu_sc as plsc

B = 4
S = 4096
D = 1024
H_POOL = 16
DPH = D // H_POOL
H_VQ = 4
DPH_VQ = D // H_VQ
K = 8192
NROW = H_VQ * B

S_CHUNK = 512
NS = S // S_CHUNK
K_CHUNK = 1024
NK = K // K_CHUNK

_DF = jax.lax.Precision.DEFAULT
_HX = jax.lax.Precision.HIGHEST


def _tc_kernel(enc_ref, wk_ref, bk_ref, wv_ref, bv_ref, cb_ref,
               dists_ref, x_ref,
               m_ref, l_ref, acc_ref, xs_ref):
    i = pl.program_id(0)

    @pl.when(i == 0)
    def _init():
        m_ref[...] = jnp.full((B, H_POOL), -jnp.inf, dtype=jnp.float32)
        l_ref[...] = jnp.zeros((B, H_POOL), dtype=jnp.float32)
        acc_ref[...] = jnp.zeros((B, H_POOL, D), dtype=jnp.float32)

    @pl.when(i < NS)
    def _pool_step():
        e = enc_ref[...]
        e2 = e.reshape(B * S_CHUNK, D)
        s = jax.lax.dot(e2, wk_ref[...],
                        precision=_DF).reshape(B, S_CHUNK, H_POOL)
        s = s + bk_ref[...][None, None, :]

        m_old = m_ref[...]
        m_new = jnp.maximum(m_old, jnp.max(s, axis=1))
        alpha = jnp.exp(m_old - m_new)
        p = jnp.exp(s - m_new[:, None, :])
        l_ref[...] = l_ref[...] * alpha + jnp.sum(p, axis=1)
        pe = jax.lax.dot_general(p, e, (((1,), (1,)), ((0,), (0,))),
                                 precision=_DF)
        acc_ref[...] = acc_ref[...] * alpha[:, :, None] + pe
        m_ref[...] = m_new

        @pl.when(i == NS - 1)
        def _finish_pool():
            pooled_e = acc_ref[...] / l_ref[...][:, :, None]
            wv_r = wv_ref[...].reshape(D, H_POOL, DPH)
            ph = jax.lax.dot_general(pooled_e, wv_r,
                                     (((2,), (0,)), ((1,), (1,))),
                                     precision=_DF)
            pooled = jnp.transpose(ph, (1, 0, 2)).reshape(B, D)
            pooled = pooled + bv_ref[...][None, :]
            xs_ref[...] = pooled
            x_hb = jnp.transpose(pooled.reshape(B, H_VQ, DPH_VQ), (1, 0, 2))
            x_ref[...] = x_hb.reshape(NROW, DPH_VQ)

    @pl.when(i >= NS)
    def _vq_step():
        x_bh = xs_ref[...].reshape(B, H_VQ, DPH_VQ)
        cb = cb_ref[...]
        xnorm = jnp.sum(x_bh * x_bh, axis=2)
        cbnorm = jnp.sum(cb * cb, axis=2)
        cross = jax.lax.dot_general(x_bh, cb, (((2,), (2,)), ((1,), (0,))),
                                    precision=_HX)
        dists = (jnp.transpose(xnorm)[:, :, None] + cbnorm[:, None, :]
                 - 2.0 * cross)
        dists_ref[...] = dists.reshape(NROW, K_CHUNK)


_SC_MESH = plsc.VectorSubcoreMesh(core_axis_name="c", subcore_axis_name="s")
_NCORE = 2
_RPC = NROW // _NCORE


def _sc_tail(dists_hbm, x_hbm, cb_hbm, q_hbm, codes_hbm, lossp_hbm,
             dv, xv, qv, rowv, idxv, kv, lv, sem):
    c = lax.axis_index("c")
    s = lax.axis_index("s")

    @pl.when(s < _RPC)
    def _():
        r = c * _RPC + s
        pltpu.sync_copy(dists_hbm.at[r], dv)

        def body(j, carry):
            vmin, vidx = carry
            v = dv[pl.ds(j * 16, 16)]
            cur = lax.iota(jnp.int32, 16) + j * 16
            better = v < vmin
            return (jnp.where(better, v, vmin),
                    jnp.where(better, cur, vidx))

        vmin0 = jnp.full((16,), jnp.inf, dtype=jnp.float32)
        vidx0 = jnp.zeros((16,), dtype=jnp.int32)
        vmin, vidx = lax.fori_loop(0, K // 16, body, (vmin0, vidx0))
        bv = vmin[0]
        bi = vidx[0]
        for l in range(1, 16):
            v = vmin[l]
            ix = vidx[l]
            take = jnp.logical_or(v < bv,
                                  jnp.logical_and(v == bv, ix < bi))
            bv = jnp.where(take, v, bv)
            bi = jnp.where(take, ix, bi)
        kbest = bi

        idxv[...] = jnp.full((16,), (r // B) * K + kbest, dtype=jnp.int32)
        pltpu.async_copy(cb_hbm.at[idxv], rowv, sem).wait()
        pltpu.sync_copy(x_hbm.at[r], xv)

        acc = jnp.zeros((16,), dtype=jnp.float32)
        for j in range(DPH_VQ // 16):
            qj = rowv[0, pl.ds(j * 16, 16)]
            xj = xv[pl.ds(j * 16, 16)]
            d = qj - xj
            qv[pl.ds(j * 16, 16)] = xj + d
            acc = acc + d * d
        pltpu.sync_copy(qv, q_hbm.at[r])
        kv[...] = jnp.full((16,), kbest, dtype=jnp.int32)
        pltpu.sync_copy(kv, codes_hbm.at[r])
        tot = acc[0]
        for l in range(1, 16):
            tot = tot + acc[l]
        lv[...] = jnp.full((16,), tot, dtype=jnp.float32)
        pltpu.sync_copy(lv, lossp_hbm.at[r])


_sc_tail_call = functools.partial(
    pl.kernel,
    out_type=[
        jax.ShapeDtypeStruct((NROW, DPH_VQ), jnp.float32),
        jax.ShapeDtypeStruct((NROW, 16), jnp.int32),
        jax.ShapeDtypeStruct((NROW, 16), jnp.float32),
    ],
    mesh=_SC_MESH,
    scratch_types=[
        pltpu.VMEM((K,), jnp.float32),
        pltpu.VMEM((DPH_VQ,), jnp.float32),
        pltpu.VMEM((DPH_VQ,), jnp.float32),
        pltpu.VMEM((16, DPH_VQ), jnp.float32),
        pltpu.VMEM((16,), jnp.int32),
        pltpu.VMEM((16,), jnp.int32),
        pltpu.VMEM((16,), jnp.float32),
        pltpu.SemaphoreType.DMA,
    ],
)(_sc_tail)


@jax.jit
def _run(encoding, Wk, bk, Wv, bv, codebooks):
    dists, x_rows = pl.pallas_call(
        _tc_kernel,
        grid=(NS + NK,),
        in_specs=[
            pl.BlockSpec((B, S_CHUNK, D),
                         lambda i: (0, jnp.minimum(i, NS - 1), 0)),
            pl.BlockSpec((D, H_POOL), lambda i: (0, 0)),
            pl.BlockSpec((H_POOL,), lambda i: (0,)),
            pl.BlockSpec((D, D), lambda i: (0, 0)),
            pl.BlockSpec((D,), lambda i: (0,)),
            pl.BlockSpec((H_VQ, K_CHUNK, DPH_VQ),
                         lambda i: (0, jnp.maximum(i - NS, 0), 0)),
        ],
        out_specs=[
            pl.BlockSpec((NROW, K_CHUNK),
                         lambda i: (0, jnp.maximum(i - NS, 0))),
            pl.BlockSpec((NROW, DPH_VQ), lambda i: (0, 0)),
        ],
        out_shape=[
            jax.ShapeDtypeStruct((NROW, K), jnp.float32),
            jax.ShapeDtypeStruct((NROW, DPH_VQ), jnp.float32),
        ],
        scratch_shapes=[
            pltpu.VMEM((B, H_POOL), jnp.float32),
            pltpu.VMEM((B, H_POOL), jnp.float32),
            pltpu.VMEM((B, H_POOL, D), jnp.float32),
            pltpu.VMEM((B, D), jnp.float32),
        ],
    )(encoding, Wk, bk, Wv, bv, codebooks)

    q_rows, codes16, lossp = _sc_tail_call(
        dists, x_rows, codebooks.reshape(H_VQ * K, DPH_VQ))

    quantized = jnp.transpose(q_rows.reshape(H_VQ, B, DPH_VQ),
                              (1, 0, 2)).reshape(B, 1, D)
    codes = jnp.transpose(codes16[:, 0].reshape(H_VQ, B))
    loss = 0.25 * jnp.sum(lossp[:, 0]) / (B * DPH_VQ)
    return quantized, loss, codes


def kernel(encoding, Wk, bk, Wv, bv, codebooks, global_step):
    del global_step
    return _run(encoding, Wk, bk, Wv, bv, codebooks)


# speedup vs baseline: 1.6445x; 1.6445x over previous
import jax
import jax.numpy as jnp
from jax.experimental import pallas as pl
from jax.experimental.pallas import tpu as pltpu

B = 4
S = 4096
D = 1024
H_POOL = 16
DPH = D // H_POOL
H_VQ = 4
DPH_VQ = D // H_VQ
K = 8192

S_CHUNK = 512
NS = S // S_CHUNK
K_CHUNK = 1024
NK = K // K_CHUNK

_DF = jax.lax.Precision.DEFAULT
_HX = jax.lax.Precision.HIGHEST


def _fused_kernel(enc_ref, wk_ref, bk_ref, wv_ref, bv_ref, cb_ref,
                  q_ref, codes_ref, loss_ref,
                  m_ref, l_ref, acc_ref, x_ref,
                  bestv_ref, besti_ref, qbest_ref):
    i = pl.program_id(0)

    @pl.when(i == 0)
    def _init():
        m_ref[...] = jnp.full((B, H_POOL), -jnp.inf, dtype=jnp.float32)
        l_ref[...] = jnp.zeros((B, H_POOL), dtype=jnp.float32)
        acc_ref[...] = jnp.zeros((B, H_POOL, D), dtype=jnp.float32)

    @pl.when(i < NS)
    def _pool_step():
        e = enc_ref[...]
        e2 = e.reshape(B * S_CHUNK, D)
        s = jax.lax.dot(e2, wk_ref[...],
                        precision=_DF).reshape(B, S_CHUNK, H_POOL)
        s = s + bk_ref[...][None, None, :]

        m_old = m_ref[...]
        m_new = jnp.maximum(m_old, jnp.max(s, axis=1))
        alpha = jnp.exp(m_old - m_new)
        p = jnp.exp(s - m_new[:, None, :])
        l_ref[...] = l_ref[...] * alpha + jnp.sum(p, axis=1)
        pe = jax.lax.dot_general(p, e, (((1,), (1,)), ((0,), (0,))),
                                 precision=_DF)
        acc_ref[...] = acc_ref[...] * alpha[:, :, None] + pe
        m_ref[...] = m_new

        @pl.when(i == NS - 1)
        def _finish_pool():
            pooled_e = acc_ref[...] / l_ref[...][:, :, None]
            wv_r = wv_ref[...].reshape(D, H_POOL, DPH)
            ph = jax.lax.dot_general(pooled_e, wv_r,
                                     (((2,), (0,)), ((1,), (1,))),
                                     precision=_DF)
            pooled = jnp.transpose(ph, (1, 0, 2)).reshape(B, D)
            x_ref[...] = pooled + bv_ref[...][None, :]

    @pl.when(i >= NS)
    def _vq_step():
        kc = i - NS

        @pl.when(kc == 0)
        def _init_vq():
            bestv_ref[...] = jnp.full((H_VQ, B), jnp.inf, dtype=jnp.float32)
            besti_ref[...] = jnp.zeros((H_VQ, B), dtype=jnp.int32)
            qbest_ref[...] = jnp.zeros((H_VQ, B, DPH_VQ), dtype=jnp.float32)

        x_bh = x_ref[...].reshape(B, H_VQ, DPH_VQ)
        cb = cb_ref[...]
        cb_hi = cb.astype(jnp.bfloat16)
        cb_lo = (cb - cb_hi.astype(jnp.float32)).astype(jnp.bfloat16)
        x_hi = x_bh.astype(jnp.bfloat16)
        x_lo = (x_bh - x_hi.astype(jnp.float32)).astype(jnp.bfloat16)
        xnorm = jnp.sum(x_bh * x_bh, axis=2)
        cbnorm = jnp.sum(cb * cb, axis=2)
        dn = (((2,), (2,)), ((1,), (0,)))
        f32 = jnp.float32
        cross = (jax.lax.dot_general(x_hi, cb_hi, dn,
                                     preferred_element_type=f32)
                 + jax.lax.dot_general(x_lo, cb_hi, dn,
                                       preferred_element_type=f32)
                 + jax.lax.dot_general(x_hi, cb_lo, dn,
                                       preferred_element_type=f32))
        dists = (jnp.transpose(xnorm)[:, :, None] + cbnorm[:, None, :]
                 - 2.0 * cross)

        cmin = jnp.min(dists, axis=2)
        ids = jax.lax.broadcasted_iota(jnp.int32, (H_VQ, B, K_CHUNK), 2)
        amin = jnp.min(jnp.where(dists == cmin[:, :, None], ids, K_CHUNK),
                       axis=2)
        onehot = (ids == amin[:, :, None]).astype(jnp.bfloat16)
        dng = (((2,), (1,)), ((0,), (0,)))
        q_cand = (jax.lax.dot_general(onehot, cb_hi, dng,
                                      preferred_element_type=f32)
                  + jax.lax.dot_general(onehot, cb_lo, dng,
                                        preferred_element_type=f32))

        better = cmin < bestv_ref[...]
        bestv_ref[...] = jnp.where(better, cmin, bestv_ref[...])
        besti_ref[...] = jnp.where(better, amin + kc * K_CHUNK,
                                   besti_ref[...])
        qbest_ref[...] = jnp.where(better[:, :, None], q_cand, qbest_ref[...])

        @pl.when(kc == NK - 1)
        def _finish():
            x_hb = jnp.transpose(x_bh, (1, 0, 2))
            q = qbest_ref[...]
            d = q - x_hb
            loss_ref[...] = (0.25 * jnp.sum(d * d) / (B * DPH_VQ)
                             ).reshape(1, 1)
            codes_ref[...] = besti_ref[...]
            q_ref[...] = x_hb + (q - x_hb)


@jax.jit
def _run(encoding, Wk, bk, Wv, bv, codebooks):
    q_hbj, codes_hb, loss = pl.pallas_call(
        _fused_kernel,
        grid=(NS + NK,),
        in_specs=[
            pl.BlockSpec((B, S_CHUNK, D),
                         lambda i: (0, jnp.minimum(i, NS - 1), 0)),
            pl.BlockSpec((D, H_POOL), lambda i: (0, 0)),
            pl.BlockSpec((H_POOL,), lambda i: (0,)),
            pl.BlockSpec((D, D), lambda i: (0, 0)),
            pl.BlockSpec((D,), lambda i: (0,)),
            pl.BlockSpec((H_VQ, K_CHUNK, DPH_VQ),
                         lambda i: (0, jnp.maximum(i - NS, 0), 0)),
        ],
        out_specs=[
            pl.BlockSpec((H_VQ, B, DPH_VQ), lambda i: (0, 0, 0)),
            pl.BlockSpec((H_VQ, B), lambda i: (0, 0)),
            pl.BlockSpec((1, 1), lambda i: (0, 0)),
        ],
        out_shape=[
            jax.ShapeDtypeStruct((H_VQ, B, DPH_VQ), jnp.float32),
            jax.ShapeDtypeStruct((H_VQ, B), jnp.int32),
            jax.ShapeDtypeStruct((1, 1), jnp.float32),
        ],
        scratch_shapes=[
            pltpu.VMEM((B, H_POOL), jnp.float32),
            pltpu.VMEM((B, H_POOL), jnp.float32),
            pltpu.VMEM((B, H_POOL, D), jnp.float32),
            pltpu.VMEM((B, D), jnp.float32),
            pltpu.VMEM((H_VQ, B), jnp.float32),
            pltpu.VMEM((H_VQ, B), jnp.int32),
            pltpu.VMEM((H_VQ, B, DPH_VQ), jnp.float32),
        ],
    )(encoding, Wk, bk, Wv, bv, codebooks)
    quantized = jnp.transpose(q_hbj, (1, 0, 2)).reshape(B, 1, D)
    return quantized, loss[0, 0], jnp.transpose(codes_hb)


def kernel(encoding, Wk, bk, Wv, bv, codebooks, global_step):
    del global_step
    return _run(encoding, Wk, bk, Wv, bv, codebooks)


# speedup vs baseline: 1.6631x; 1.0113x over previous
import jax
import jax.numpy as jnp
from jax.experimental import pallas as pl
from jax.experimental.pallas import tpu as pltpu

B = 4
S = 4096
D = 1024
H_POOL = 16
DPH = D // H_POOL
H_VQ = 4
DPH_VQ = D // H_VQ
K = 8192

S_CHUNK = 512
NS = S // S_CHUNK
K_CHUNK = 2048
NK = K // K_CHUNK

_DF = jax.lax.Precision.DEFAULT
_HX = jax.lax.Precision.HIGHEST


def _fused_kernel(enc_ref, wk_ref, bk_ref, wv_ref, bv_ref, cb_ref,
                  q_ref, codes_ref, loss_ref,
                  m_ref, l_ref, acc_ref, x_ref,
                  bestv_ref, besti_ref, qbest_ref):
    i = pl.program_id(0)

    @pl.when(i == 0)
    def _init():
        m_ref[...] = jnp.full((B, H_POOL), -jnp.inf, dtype=jnp.float32)
        l_ref[...] = jnp.zeros((B, H_POOL), dtype=jnp.float32)
        acc_ref[...] = jnp.zeros((B, H_POOL, D), dtype=jnp.float32)

    @pl.when(i < NS)
    def _pool_step():
        e = enc_ref[...]
        e2 = e.reshape(B * S_CHUNK, D)
        s = jax.lax.dot(e2, wk_ref[...],
                        precision=_DF).reshape(B, S_CHUNK, H_POOL)
        s = s + bk_ref[...][None, None, :]

        m_old = m_ref[...]
        m_new = jnp.maximum(m_old, jnp.max(s, axis=1))
        alpha = jnp.exp(m_old - m_new)
        p = jnp.exp(s - m_new[:, None, :])
        l_ref[...] = l_ref[...] * alpha + jnp.sum(p, axis=1)
        pe = jax.lax.dot_general(p, e, (((1,), (1,)), ((0,), (0,))),
                                 precision=_DF)
        acc_ref[...] = acc_ref[...] * alpha[:, :, None] + pe
        m_ref[...] = m_new

        @pl.when(i == NS - 1)
        def _finish_pool():
            pooled_e = acc_ref[...] / l_ref[...][:, :, None]
            wv_r = wv_ref[...].reshape(D, H_POOL, DPH)
            ph = jax.lax.dot_general(pooled_e, wv_r,
                                     (((2,), (0,)), ((1,), (1,))),
                                     precision=_DF)
            pooled = jnp.transpose(ph, (1, 0, 2)).reshape(B, D)
            x_ref[...] = pooled + bv_ref[...][None, :]

    @pl.when(i >= NS)
    def _vq_step():
        kc = i - NS

        @pl.when(kc == 0)
        def _init_vq():
            bestv_ref[...] = jnp.full((H_VQ, B), jnp.inf, dtype=jnp.float32)
            besti_ref[...] = jnp.zeros((H_VQ, B), dtype=jnp.int32)
            qbest_ref[...] = jnp.zeros((H_VQ, B, DPH_VQ), dtype=jnp.float32)

        x_bh = x_ref[...].reshape(B, H_VQ, DPH_VQ)
        cb = cb_ref[...]
        cb_hi = cb.astype(jnp.bfloat16)
        cb_lo = (cb - cb_hi.astype(jnp.float32)).astype(jnp.bfloat16)
        x_hi = x_bh.astype(jnp.bfloat16)
        x_lo = (x_bh - x_hi.astype(jnp.float32)).astype(jnp.bfloat16)
        xnorm = jnp.sum(x_bh * x_bh, axis=2)
        cbnorm = jnp.sum(cb * cb, axis=2)
        dn = (((2,), (2,)), ((1,), (0,)))
        f32 = jnp.float32
        cross = (jax.lax.dot_general(x_hi, cb_hi, dn,
                                     preferred_element_type=f32)
                 + jax.lax.dot_general(x_lo, cb_hi, dn,
                                       preferred_element_type=f32)
                 + jax.lax.dot_general(x_hi, cb_lo, dn,
                                       preferred_element_type=f32))
        dists = (jnp.transpose(xnorm)[:, :, None] + cbnorm[:, None, :]
                 - 2.0 * cross)

        cmin = jnp.min(dists, axis=2)
        ids = jax.lax.broadcasted_iota(jnp.int32, (H_VQ, B, K_CHUNK), 2)
        amin = jnp.min(jnp.where(dists == cmin[:, :, None], ids, K_CHUNK),
                       axis=2)
        onehot = (ids == amin[:, :, None]).astype(jnp.bfloat16)
        dng = (((2,), (1,)), ((0,), (0,)))
        q_cand = (jax.lax.dot_general(onehot, cb_hi, dng,
                                      preferred_element_type=f32)
                  + jax.lax.dot_general(onehot, cb_lo, dng,
                                        preferred_element_type=f32))

        better = cmin < bestv_ref[...]
        bestv_ref[...] = jnp.where(better, cmin, bestv_ref[...])
        besti_ref[...] = jnp.where(better, amin + kc * K_CHUNK,
                                   besti_ref[...])
        qbest_ref[...] = jnp.where(better[:, :, None], q_cand, qbest_ref[...])

        @pl.when(kc == NK - 1)
        def _finish():
            x_hb = jnp.transpose(x_bh, (1, 0, 2))
            q = qbest_ref[...]
            d = q - x_hb
            loss_ref[...] = (0.25 * jnp.sum(d * d) / (B * DPH_VQ)
                             ).reshape(1, 1)
            codes_ref[...] = besti_ref[...]
            q_ref[...] = x_hb + (q - x_hb)


@jax.jit
def _run(encoding, Wk, bk, Wv, bv, codebooks):
    q_hbj, codes_hb, loss = pl.pallas_call(
        _fused_kernel,
        grid=(NS + NK,),
        in_specs=[
            pl.BlockSpec((B, S_CHUNK, D),
                         lambda i: (0, jnp.minimum(i, NS - 1), 0)),
            pl.BlockSpec((D, H_POOL), lambda i: (0, 0)),
            pl.BlockSpec((H_POOL,), lambda i: (0,)),
            pl.BlockSpec((D, D), lambda i: (0, 0)),
            pl.BlockSpec((D,), lambda i: (0,)),
            pl.BlockSpec((H_VQ, K_CHUNK, DPH_VQ),
                         lambda i: (0, jnp.maximum(i - NS, 0), 0)),
        ],
        out_specs=[
            pl.BlockSpec((H_VQ, B, DPH_VQ), lambda i: (0, 0, 0)),
            pl.BlockSpec((H_VQ, B), lambda i: (0, 0)),
            pl.BlockSpec((1, 1), lambda i: (0, 0)),
        ],
        out_shape=[
            jax.ShapeDtypeStruct((H_VQ, B, DPH_VQ), jnp.float32),
            jax.ShapeDtypeStruct((H_VQ, B), jnp.int32),
            jax.ShapeDtypeStruct((1, 1), jnp.float32),
        ],
        scratch_shapes=[
            pltpu.VMEM((B, H_POOL), jnp.float32),
            pltpu.VMEM((B, H_POOL), jnp.float32),
            pltpu.VMEM((B, H_POOL, D), jnp.float32),
            pltpu.VMEM((B, D), jnp.float32),
            pltpu.VMEM((H_VQ, B), jnp.float32),
            pltpu.VMEM((H_VQ, B), jnp.int32),
            pltpu.VMEM((H_VQ, B, DPH_VQ), jnp.float32),
        ],
    )(encoding, Wk, bk, Wv, bv, codebooks)
    quantized = jnp.transpose(q_hbj, (1, 0, 2)).reshape(B, 1, D)
    return quantized, loss[0, 0], jnp.transpose(codes_hb)


def kernel(encoding, Wk, bk, Wv, bv, codebooks, global_step):
    del global_step
    return _run(encoding, Wk, bk, Wv, bv, codebooks)
